# baseline (speedup 1.0000x reference)
import jax
import jax.numpy as jnp
from jax import lax
from jax.experimental import pallas as pl
from jax.experimental.pallas import tpu as pltpu
from jax.experimental.pallas import tpu_sc as plsc

VOCAB = 100000
EMBED = 50
EMBED_P = 56
BATCH = 4096
SEQ = 50

NUM_CORES = 2
NUM_SUBCORES = 16
NUM_WORKERS = NUM_CORES * NUM_SUBCORES

BPT = BATCH // NUM_WORKERS
G = 8
NGROUPS = BPT // G
GROWS = G * SEQ
GWORDS = GROWS * EMBED
WPT = BPT * SEQ * EMBED


def _gather_body(idx_hbm, table_hbm, out_hbm, idx_v, gbuf, pbuf,
                 sem_g0, sem_g1, sem_o0, sem_o1):
    cid = lax.axis_index("c")
    sid = lax.axis_index("s")
    wid = sid * NUM_CORES + cid
    b0 = wid * BPT
    w0 = wid * WPT
    sem_g = (sem_g0, sem_g1)
    sem_o = (sem_o0, sem_o1)

    pltpu.sync_copy(idx_hbm.at[pl.ds(b0, BPT), :], idx_v)

    def fire(g, p):
        for k in range(G):
            pltpu.async_copy(
                table_hbm.at[idx_v.at[g * G + k, :]],
                gbuf.at[p, pl.ds(k * SEQ, SEQ), :],
                sem_g[p],
            )

    def drain(p):
        for k in range(G):
            pltpu.make_async_copy(
                table_hbm.at[idx_v.at[k, :]],
                gbuf.at[p, pl.ds(k * SEQ, SEQ), :],
                sem_g[p],
            ).wait()

    def repack(p):
        def row(r):
            for c in (0, 16, 32, 34):
                pbuf[p, pl.ds(r * EMBED + c, 16)] = gbuf[p, r, pl.ds(c, 16)]

        pl.loop(0, GROWS, unroll=8)(row)

    def writeout(g, p):
        pltpu.async_copy(
            pbuf.at[p], out_hbm.at[pl.ds(w0 + g * GWORDS, GWORDS)], sem_o[p]
        )

    def waitout(g, p):
        pltpu.make_async_copy(
            pbuf.at[p], out_hbm.at[pl.ds(w0 + g * GWORDS, GWORDS)], sem_o[p]
        ).wait()

    fire(0, 0)

    def step(i):
        g0 = 2 * i
        g1 = 2 * i + 1

        @pl.when(g1 < NGROUPS)
        def _():
            fire(g1, 1)
        drain(0)
        repack(0)
        writeout(g0, 0)

        @pl.when(g1 < NGROUPS)
        def _():
            drain(1)

            @pl.when(g1 + 1 < NGROUPS)
            def _():
                fire(g1 + 1, 0)
            repack(1)
            writeout(g1, 1)
            waitout(g1, 1)
        waitout(g0, 0)

    pl.loop(0, (NGROUPS + 1) // 2)(step)


@jax.jit
def _run(idx, table_p):
    kern = pl.kernel(
        _gather_body,
        out_type=jax.ShapeDtypeStruct((BATCH * SEQ * EMBED,), jnp.float32),
        mesh=plsc.VectorSubcoreMesh(core_axis_name="c", subcore_axis_name="s"),
        scratch_types=[
            pltpu.VMEM((BPT, SEQ), jnp.int32),
            pltpu.VMEM((2, GROWS, EMBED_P), jnp.float32),
            pltpu.VMEM((2, GWORDS), jnp.float32),
            pltpu.SemaphoreType.DMA,
            pltpu.SemaphoreType.DMA,
            pltpu.SemaphoreType.DMA,
            pltpu.SemaphoreType.DMA,
        ],
        compiler_params=pltpu.CompilerParams(use_tc_tiling_on_sc=False),
    )
    return kern(idx, table_p)


def kernel(indices, table):
    idx = indices.astype(jnp.int32)
    table_p = jnp.pad(table, ((0, 0), (0, EMBED_P - EMBED)))
    out = _run(idx, table_p)
    return out.reshape(BATCH, SEQ, EMBED)


# speedup vs baseline: 1.2412x; 1.2412x over previous
import jax
import jax.numpy as jnp
from jax import lax
from jax.experimental import pallas as pl
from jax.experimental.pallas import tpu as pltpu
from jax.experimental.pallas import tpu_sc as plsc

VOCAB = 100000
EMBED = 50
EMBED_P = 56
BATCH = 4096
SEQ = 50

NUM_CORES = 2
NUM_SUBCORES = 16
NUM_WORKERS = NUM_CORES * NUM_SUBCORES

BPT = BATCH // NUM_WORKERS
G = 8
NGROUPS = BPT // G
GROWS = G * SEQ


def _gather_body(idx_hbm, table_hbm, out_hbm, idx_v, gbuf, pbuf,
                 sem_g0, sem_g1, sem_o0, sem_o1):
    cid = lax.axis_index("c")
    sid = lax.axis_index("s")
    wid = sid * NUM_CORES + cid
    b0 = wid * BPT
    sem_g = (sem_g0, sem_g1)
    sem_o = (sem_o0, sem_o1)

    pltpu.sync_copy(idx_hbm.at[pl.ds(b0, BPT), :], idx_v)

    def fire(g, p):
        for k in range(G):
            pltpu.async_copy(
                table_hbm.at[idx_v.at[g * G + k, :]],
                gbuf.at[p, pl.ds(k * SEQ, SEQ), :],
                sem_g[p],
            )

    def drain(p):
        for k in range(G):
            pltpu.make_async_copy(
                table_hbm.at[idx_v.at[k, :]],
                gbuf.at[p, pl.ds(k * SEQ, SEQ), :],
                sem_g[p],
            ).wait()

    def repack(p):
        def row(r):
            k = r // SEQ
            j = r - k * SEQ
            for c in (0, 16, 32, 34):
                pbuf[p, k, j, pl.ds(c, 16)] = gbuf[p, r, pl.ds(c, 16)]

        pl.loop(0, GROWS, unroll=8)(row)

    def writeout(g, p):
        pltpu.async_copy(
            pbuf.at[p], out_hbm.at[pl.ds(b0 + g * G, G), :, :], sem_o[p]
        )

    def waitout(g, p):
        pltpu.make_async_copy(
            pbuf.at[p], out_hbm.at[pl.ds(b0 + g * G, G), :, :], sem_o[p]
        ).wait()

    fire(0, 0)

    def step(i):
        g0 = 2 * i
        g1 = 2 * i + 1

        @pl.when(g1 < NGROUPS)
        def _():
            fire(g1, 1)
        drain(0)
        repack(0)
        writeout(g0, 0)

        @pl.when(g1 < NGROUPS)
        def _():
            drain(1)

            @pl.when(g1 + 1 < NGROUPS)
            def _():
                fire(g1 + 1, 0)
            repack(1)
            writeout(g1, 1)
            waitout(g1, 1)
        waitout(g0, 0)

    pl.loop(0, (NGROUPS + 1) // 2)(step)


@jax.jit
def _run(idx, table_p):
    kern = pl.kernel(
        _gather_body,
        out_type=jax.ShapeDtypeStruct((BATCH, SEQ, EMBED), jnp.float32),
        mesh=plsc.VectorSubcoreMesh(core_axis_name="c", subcore_axis_name="s"),
        scratch_types=[
            pltpu.VMEM((BPT, SEQ), jnp.int32),
            pltpu.VMEM((2, GROWS, EMBED_P), jnp.float32),
            pltpu.VMEM((2, G, SEQ, EMBED), jnp.float32),
            pltpu.SemaphoreType.DMA,
            pltpu.SemaphoreType.DMA,
            pltpu.SemaphoreType.DMA,
            pltpu.SemaphoreType.DMA,
        ],
        compiler_params=pltpu.CompilerParams(use_tc_tiling_on_sc=False),
    )
    return kern(idx, table_p)


def kernel(indices, table):
    idx = indices.astype(jnp.int32)
    table_p = jnp.pad(table, ((0, 0), (0, EMBED_P - EMBED)))
    return _run(idx, table_p)


# speedup vs baseline: 1.2583x; 1.0138x over previous
import jax
import jax.numpy as jnp
from jax import lax
from jax.experimental import pallas as pl
from jax.experimental.pallas import tpu as pltpu
from jax.experimental.pallas import tpu_sc as plsc

VOCAB = 100000
EMBED = 50
EMBED_P = 56
BATCH = 4096
SEQ = 50

NUM_CORES = 2
NUM_SUBCORES = 16
NUM_WORKERS = NUM_CORES * NUM_SUBCORES

BPT = BATCH // NUM_WORKERS
G = 8
NGROUPS = BPT // G
GROWS = G * SEQ


def _gather_body(idx_hbm, table_hbm, out_hbm, idx_v, gbuf, pbuf,
                 sem_g0, sem_g1, sem_o0, sem_o1):
    cid = lax.axis_index("c")
    sid = lax.axis_index("s")
    wid = sid * NUM_CORES + cid
    b0 = wid * BPT
    sem_g = (sem_g0, sem_g1)
    sem_o = (sem_o0, sem_o1)

    pltpu.sync_copy(idx_hbm.at[pl.ds(b0, BPT), :], idx_v)

    def fire(g, p):
        for k in range(G):
            pltpu.async_copy(
                table_hbm.at[idx_v.at[g * G + k, :]],
                gbuf.at[p, pl.ds(k * SEQ, SEQ), :],
                sem_g[p],
            )

    def drain(p):
        for k in range(G):
            pltpu.make_async_copy(
                table_hbm.at[idx_v.at[k, :]],
                gbuf.at[p, pl.ds(k * SEQ, SEQ), :],
                sem_g[p],
            ).wait()

    def repack(p):
        for k in range(G):
            def row(j, _k=k):
                for c in (0, 16, 32, 34):
                    pbuf[p, _k, j, pl.ds(c, 16)] = gbuf[p, _k * SEQ + j, pl.ds(c, 16)]

            pl.loop(0, SEQ, unroll=10)(row)

    def writeout(g, p):
        pltpu.async_copy(
            pbuf.at[p], out_hbm.at[pl.ds(b0 + g * G, G), :, :], sem_o[p]
        )

    def waitout(g, p):
        pltpu.make_async_copy(
            pbuf.at[p], out_hbm.at[pl.ds(b0 + g * G, G), :, :], sem_o[p]
        ).wait()

    fire(0, 0)

    def step(i):
        g0 = 2 * i
        g1 = 2 * i + 1

        @pl.when(g1 < NGROUPS)
        def _():
            fire(g1, 1)
        drain(0)
        repack(0)
        writeout(g0, 0)

        @pl.when(g1 < NGROUPS)
        def _():
            drain(1)

            @pl.when(g1 + 1 < NGROUPS)
            def _():
                fire(g1 + 1, 0)
            repack(1)
            writeout(g1, 1)
            waitout(g1, 1)
        waitout(g0, 0)

    pl.loop(0, (NGROUPS + 1) // 2)(step)


@jax.jit
def _run(idx, table_p):
    kern = pl.kernel(
        _gather_body,
        out_type=jax.ShapeDtypeStruct((BATCH, SEQ, EMBED), jnp.float32),
        mesh=plsc.VectorSubcoreMesh(core_axis_name="c", subcore_axis_name="s"),
        scratch_types=[
            pltpu.VMEM((BPT, SEQ), jnp.int32),
            pltpu.VMEM((2, GROWS, EMBED_P), jnp.float32),
            pltpu.VMEM((2, G, SEQ, EMBED), jnp.float32),
            pltpu.SemaphoreType.DMA,
            pltpu.SemaphoreType.DMA,
            pltpu.SemaphoreType.DMA,
            pltpu.SemaphoreType.DMA,
        ],
        compiler_params=pltpu.CompilerParams(use_tc_tiling_on_sc=False),
    )
    return kern(idx, table_p)


def kernel(indices, table):
    idx = indices.astype(jnp.int32)
    table_p = jnp.pad(table, ((0, 0), (0, EMBED_P - EMBED)))
    return _run(idx, table_p)


# speedup vs baseline: 1.3744x; 1.0922x over previous
import jax
import jax.numpy as jnp
from jax import lax
from jax.experimental import pallas as pl
from jax.experimental.pallas import tpu as pltpu
from jax.experimental.pallas import tpu_sc as plsc

VOCAB = 100000
EMBED = 50
EMBED_P = 56
BATCH = 4096
SEQ = 50

NUM_CORES = 2
NUM_SUBCORES = 16
NUM_WORKERS = NUM_CORES * NUM_SUBCORES

BPT = BATCH // NUM_WORKERS
G = 8
NGROUPS = BPT // G


def _gather_body(idx_hbm, table_hbm, out_hbm, idx_v, buf,
                 sem_g0, sem_g1, sem_o0, sem_o1):
    cid = lax.axis_index("c")
    sid = lax.axis_index("s")
    wid = sid * NUM_CORES + cid
    b0 = wid * BPT
    sem_g = (sem_g0, sem_g1)
    sem_o = (sem_o0, sem_o1)

    pltpu.sync_copy(idx_hbm.at[pl.ds(b0, BPT), :], idx_v)

    def fire(g, p):
        for k in range(G):
            pltpu.async_copy(
                table_hbm.at[idx_v.at[g * G + k, :]],
                buf.at[p, k],
                sem_g[p],
            )

    def drain(p):
        for k in range(G):
            pltpu.make_async_copy(
                table_hbm.at[idx_v.at[k, :]],
                buf.at[p, k],
                sem_g[p],
            ).wait()

    def writeout(g, p):
        pltpu.async_copy(
            buf.at[p], out_hbm.at[pl.ds(b0 + g * G, G), :, :], sem_o[p]
        )

    def waitout(g, p):
        pltpu.make_async_copy(
            buf.at[p], out_hbm.at[pl.ds(b0 + g * G, G), :, :], sem_o[p]
        ).wait()

    fire(0, 0)

    def step(i):
        g0 = 2 * i
        g1 = 2 * i + 1

        @pl.when(g1 < NGROUPS)
        def _():
            fire(g1, 1)
        drain(0)
        writeout(g0, 0)

        @pl.when(g1 < NGROUPS)
        def _():
            drain(1)

            @pl.when(g1 + 1 < NGROUPS)
            def _():
                fire(g1 + 1, 0)
            writeout(g1, 1)
            waitout(g1, 1)
        waitout(g0, 0)

    pl.loop(0, (NGROUPS + 1) // 2)(step)


@jax.jit
def _run(idx, table_p):
    kern = pl.kernel(
        _gather_body,
        out_type=jax.ShapeDtypeStruct((BATCH, SEQ, EMBED_P), jnp.float32),
        mesh=plsc.VectorSubcoreMesh(core_axis_name="c", subcore_axis_name="s"),
        scratch_types=[
            pltpu.VMEM((BPT, SEQ), jnp.int32),
            pltpu.VMEM((2, G, SEQ, EMBED_P), jnp.float32),
            pltpu.SemaphoreType.DMA,
            pltpu.SemaphoreType.DMA,
            pltpu.SemaphoreType.DMA,
            pltpu.SemaphoreType.DMA,
        ],
        compiler_params=pltpu.CompilerParams(use_tc_tiling_on_sc=False),
    )
    return kern(idx, table_p)


def kernel(indices, table):
    idx = indices.astype(jnp.int32)
    table_p = jnp.pad(table, ((0, 0), (0, EMBED_P - EMBED)))
    out = _run(idx, table_p)
    return out[:, :, :EMBED]
